# baseline (speedup 1.0000x reference)
import functools

import jax
import jax.numpy as jnp
from jax.experimental import pallas as pl

_FEATS = (160, 160, 128, 128)


def _ll_kernel(sp_ref, y_ref, wc_ref, c_ref, st_ref, out_ref, *, n_elem,
               inv_nets):
    out = jnp.dot(y_ref[...], wc_ref[...], preferred_element_type=jnp.float32,
                  precision=jax.lax.Precision.HIGHEST)
    eidx = jax.lax.broadcasted_iota(jnp.int32, (out.shape[0], n_elem), 1)
    onehot = (sp_ref[...] == eidx).astype(jnp.float32)
    sel = jnp.sum(onehot * (out * inv_nets + c_ref[...]), axis=1,
                  keepdims=True)
    out_ref[...] = jnp.dot(st_ref[...], sel,
                           preferred_element_type=jnp.float32,
                           precision=jax.lax.Precision.HIGHEST)


@jax.jit
def kernel(species, y, W, b, self_energies):
    B, A, NETS, F = y.shape
    N_ELEM = W.shape[1]
    KF = NETS * F

    feats = jnp.asarray(_FEATS[:N_ELEM], dtype=jnp.int32)
    fmask = (jnp.arange(F, dtype=jnp.int32)[None, :] < feats[:, None])
    Wm = W * fmask[None, :, :].astype(W.dtype)
    Wc = Wm.transpose(0, 2, 1).reshape(KF, N_ELEM)
    c = (b.sum(axis=0) / NETS + self_energies)[None, :]

    BM = 64
    NA = BM * A
    y2 = y.reshape(B * A, KF)
    sp2 = species.reshape(B * A, 1)
    st = (jnp.arange(NA, dtype=jnp.int32)[None, :] // A
          == jnp.arange(BM, dtype=jnp.int32)[:, None]).astype(jnp.float32)

    grid = (B // BM,)
    out = pl.pallas_call(
        functools.partial(_ll_kernel, n_elem=N_ELEM, inv_nets=1.0 / NETS),
        grid=grid,
        in_specs=[
            pl.BlockSpec((NA, 1), lambda m: (m, 0)),
            pl.BlockSpec((NA, KF), lambda m: (m, 0)),
            pl.BlockSpec((KF, N_ELEM), lambda m: (0, 0)),
            pl.BlockSpec((1, N_ELEM), lambda m: (0, 0)),
            pl.BlockSpec((BM, NA), lambda m: (0, 0)),
        ],
        out_specs=pl.BlockSpec((BM, 1), lambda m: (m, 0)),
        out_shape=jax.ShapeDtypeStruct((B, 1), jnp.float32),
    )(sp2, y2, Wc, c, st)

    return (species, out.reshape(B))


# speedup vs baseline: 1.2100x; 1.2100x over previous
import functools

import jax
import jax.numpy as jnp
from jax.experimental import pallas as pl

_FEATS = (160, 160, 128, 128)


def _ll_kernel(sp_ref, y_ref, wc_ref, c_ref, st_ref, out_ref, *, n_elem,
               inv_nets):
    out = jnp.dot(y_ref[...], wc_ref[...], preferred_element_type=jnp.float32,
                  precision=jax.lax.Precision.DEFAULT)
    eidx = jax.lax.broadcasted_iota(jnp.int32, (out.shape[0], n_elem), 1)
    onehot = (sp_ref[...] == eidx).astype(jnp.float32)
    sel = jnp.sum(onehot * (out * inv_nets + c_ref[...]), axis=1,
                  keepdims=True)
    out_ref[...] = jnp.dot(st_ref[...], sel,
                           preferred_element_type=jnp.float32,
                           precision=jax.lax.Precision.HIGHEST)


@jax.jit
def kernel(species, y, W, b, self_energies):
    B, A, NETS, F = y.shape
    N_ELEM = W.shape[1]
    KF = NETS * F

    feats = jnp.asarray(_FEATS[:N_ELEM], dtype=jnp.int32)
    fmask = (jnp.arange(F, dtype=jnp.int32)[None, :] < feats[:, None])
    Wm = W * fmask[None, :, :].astype(W.dtype)
    Wc = Wm.transpose(0, 2, 1).reshape(KF, N_ELEM)
    c = (b.sum(axis=0) / NETS + self_energies)[None, :]

    BM = 64
    NA = BM * A
    y2 = y.reshape(B * A, KF)
    sp2 = species.reshape(B * A, 1)
    st = (jnp.arange(NA, dtype=jnp.int32)[None, :] // A
          == jnp.arange(BM, dtype=jnp.int32)[:, None]).astype(jnp.float32)

    grid = (B // BM,)
    out = pl.pallas_call(
        functools.partial(_ll_kernel, n_elem=N_ELEM, inv_nets=1.0 / NETS),
        grid=grid,
        in_specs=[
            pl.BlockSpec((NA, 1), lambda m: (m, 0)),
            pl.BlockSpec((NA, KF), lambda m: (m, 0)),
            pl.BlockSpec((KF, N_ELEM), lambda m: (0, 0)),
            pl.BlockSpec((1, N_ELEM), lambda m: (0, 0)),
            pl.BlockSpec((BM, NA), lambda m: (0, 0)),
        ],
        out_specs=pl.BlockSpec((BM, 1), lambda m: (m, 0)),
        out_shape=jax.ShapeDtypeStruct((B, 1), jnp.float32),
    )(sp2, y2, Wc, c, st)

    return (species, out.reshape(B))


# speedup vs baseline: 1.5564x; 1.2862x over previous
import functools

import jax
import jax.numpy as jnp
from jax.experimental import pallas as pl

_FEATS = (160, 160, 128, 128)


def _ll_kernel(sp_ref, y_ref, wm_ref, c_ref, out_ref, *, n_atoms, n_elem,
               inv_nets):
    n_nets = y_ref.shape[2]
    out = jnp.zeros((y_ref.shape[0], n_atoms, n_elem), dtype=jnp.float32)
    for i in range(n_nets):
        out = out + jax.lax.dot_general(
            y_ref[:, :, i, :], wm_ref[i],
            dimension_numbers=(((2,), (0,)), ((), ())),
            preferred_element_type=jnp.float32)
    sp = sp_ref[...]
    eidx = jax.lax.broadcasted_iota(jnp.int32, out.shape, 2)
    onehot = (sp[:, :, None] == eidx).astype(jnp.float32)
    contrib = jnp.sum(onehot * ((out + c_ref[...][None, :, :]) * inv_nets),
                      axis=2)
    out_ref[...] = jnp.sum(contrib, axis=1, keepdims=True)


@jax.jit
def kernel(species, y, W, b, self_energies):
    B, A, NETS, F = y.shape
    N_ELEM = W.shape[1]

    feats = jnp.asarray(_FEATS[:N_ELEM], dtype=jnp.int32)
    fmask = (jnp.arange(F, dtype=jnp.int32)[None, :] < feats[:, None])
    Wm = (W * fmask[None, :, :].astype(W.dtype)).transpose(0, 2, 1)
    c = (b.sum(axis=0) + self_energies * NETS)[None, :]

    BM = 64
    grid = (B // BM,)
    out = pl.pallas_call(
        functools.partial(_ll_kernel, n_atoms=A, n_elem=N_ELEM,
                          inv_nets=1.0 / NETS),
        grid=grid,
        in_specs=[
            pl.BlockSpec((BM, A), lambda m: (m, 0)),
            pl.BlockSpec((BM, A, NETS, F), lambda m: (m, 0, 0, 0)),
            pl.BlockSpec((NETS, F, N_ELEM), lambda m: (0, 0, 0)),
            pl.BlockSpec((1, N_ELEM), lambda m: (0, 0)),
        ],
        out_specs=pl.BlockSpec((BM, 1), lambda m: (m, 0)),
        out_shape=jax.ShapeDtypeStruct((B, 1), jnp.float32),
    )(species, y, Wm, c)

    return (species, out.reshape(B))
